# Initial kernel scaffold; baseline (speedup 1.0000x reference)
import jax
import jax.numpy as jnp
from jax.experimental import pallas as pl


def kernel(x, pos, norm, edge_index, W1, b1, g1, be1, W2, b2, g2, be2, W3, b3):
    raise NotImplementedError("write your pallas kernel here")


# speedup vs baseline: 3.8545x; 3.8545x over previous
import dataclasses
import functools

import jax
import jax.numpy as jnp
from jax import lax
from jax.experimental import pallas as pl
from jax.experimental.pallas import tpu as pltpu
from jax.experimental.pallas import tpu_sc as plsc

_N = 10000
_NP = 10240
_E = 330000
_BLK = 2048
_NBLK = 162
_EP = _BLK * _NBLK
_NW = 32
_EW = _EP // _NW
_CH = 128
_NCH = _EW // _CH
_TR = _EP // 128
_ROWS_W = _NP // _NW
_ACC_R = _ROWS_W + 8
_CAP = 16384
_EPS = 1e-5
_NEG = -3.0e38

_sc_params = pltpu.CompilerParams()
if "needs_layout_passes" in pltpu.CompilerParams.__dataclass_fields__:
    _sc_params = dataclasses.replace(_sc_params, needs_layout_passes=False)
if "use_tc_tiling_on_sc" in pltpu.CompilerParams.__dataclass_fields__:
    _sc_params = dataclasses.replace(_sc_params, use_tc_tiling_on_sc=False)


def _u_body(x_ref, w_ref, o_ref):
    o_ref[...] = jnp.dot(x_ref[...], w_ref[...],
                         preferred_element_type=jnp.float32)


def _node_proj(x, w1x):
    return pl.pallas_call(
        _u_body,
        out_shape=jax.ShapeDtypeStruct((_N, 32), jnp.float32),
    )(x, w1x)


@functools.lru_cache(maxsize=None)
def _build_gather_sc():
    mesh = plsc.VectorSubcoreMesh(core_axis_name="c", subcore_axis_name="s")

    @functools.partial(
        pl.kernel,
        mesh=mesh,
        out_type=[
            jax.ShapeDtypeStruct((12, _TR, 128), jnp.float32),
            jax.ShapeDtypeStruct((_EP, 32), jnp.float32),
        ],
        scratch_types=[
            pltpu.VMEM((_NP * 8,), jnp.float32),
            pltpu.VMEM((_CH,), jnp.int32),
            pltpu.VMEM((_CH,), jnp.int32),
            pltpu.VMEM((12, 1, 128), jnp.float32),
            pltpu.VMEM((_CH, 32), jnp.float32),
            pltpu.SemaphoreType.DMA,
        ],
        compiler_params=_sc_params,
    )
    def _gather_sc(g_hbm, u_hbm, src_hbm, dst_hbm, comp_hbm, us_hbm,
                   gv, sv, dv, cb, ub, sem):
        wid = lax.axis_index("s") * 2 + lax.axis_index("c")
        base = wid * _EW
        pltpu.sync_copy(g_hbm, gv)

        @pl.loop(0, _NCH)
        def _chunk(t):
            eb = base + t * _CH
            pltpu.sync_copy(src_hbm.at[pl.ds(eb, _CH)], sv)
            pltpu.sync_copy(dst_hbm.at[pl.ds(eb, _CH)], dv)
            h = pltpu.async_copy(u_hbm.at[sv], ub, sem)

            @pl.loop(0, _CH // 16)
            def _vec(v):
                si = sv[pl.ds(v * 16, 16)] * 8
                di = dv[pl.ds(v * 16, 16)] * 8
                for k in range(6):
                    cb[k, 0, pl.ds(v * 16, 16)] = plsc.load_gather(gv, [si + k])
                    cb[6 + k, 0, pl.ds(v * 16, 16)] = plsc.load_gather(
                        gv, [di + k])

            pltpu.sync_copy(cb, comp_hbm.at[:, pl.ds(eb // 128, 1), :])
            h.wait()
            pltpu.sync_copy(ub, us_hbm.at[pl.ds(eb, _CH)])

    return _gather_sc


def _safenorm(x, y, z):
    sq = x * x + y * y + z * z
    return jnp.where(sq > 0, jnp.sqrt(jnp.where(sq > 0, sq, 1.0)), 0.0)


def _angle(ax, ay, az, bx, by, bz):
    cx = ay * bz - az * by
    cy = az * bx - ax * bz
    cz = ax * by - ay * bx
    dot = ax * bx + ay * by + az * bz
    dot = jnp.where(dot == 0.0, 0.0, dot)
    return jnp.arctan2(_safenorm(cx, cy, cz), dot)


def _ppf_body(c_ref, o_ref):
    c = c_ref[...]
    pjx, pjy, pjz = c[0], c[1], c[2]
    njx, njy, njz = c[3], c[4], c[5]
    pix, piy, piz = c[6], c[7], c[8]
    nix, niy, niz = c[9], c[10], c[11]
    dx, dy, dz = pjx - pix, pjy - piy, pjz - piz
    f1 = _safenorm(dx, dy, dz)
    f2 = _angle(nix, niy, niz, dx, dy, dz)
    f3 = _angle(njx, njy, njz, dx, dy, dz)
    f4 = _angle(nix, niy, niz, njx, njy, njz)
    z = jnp.zeros_like(f1)
    o_ref[...] = jnp.stack([f1, f2, f3, f4, z, z, z, z], axis=0)


def _ppf(comp):
    return pl.pallas_call(
        _ppf_body,
        grid=(_TR // 16,),
        in_specs=[pl.BlockSpec((12, 16, 128), lambda i: (0, i, 0))],
        out_specs=pl.BlockSpec((8, 16, 128), lambda i: (0, i, 0)),
        out_shape=jax.ShapeDtypeStruct((8, _TR, 128), jnp.float32),
    )(comp)


def _b1_body(us_ref, pf_ref, w_ref, b_ref, h_ref, st_ref):
    i = pl.program_id(0)

    @pl.when(i == 0)
    def _():
        st_ref[...] = jnp.zeros_like(st_ref)

    h = us_ref[...] + jnp.dot(pf_ref[...], w_ref[...],
                              preferred_element_type=jnp.float32) + b_ref[...]
    h_ref[...] = h
    gid = i * _BLK + lax.broadcasted_iota(jnp.int32, (_BLK, 1), 0)
    m = (gid < _E).astype(jnp.float32)
    hm = h * m
    st_ref[0:1, 0:32] = st_ref[0:1, 0:32] + jnp.sum(hm, axis=0, keepdims=True)
    st_ref[1:2, 0:32] = st_ref[1:2, 0:32] + jnp.sum(hm * h, axis=0, keepdims=True)


def _b1(us, pf, w1p, b1):
    return pl.pallas_call(
        _b1_body,
        grid=(_NBLK,),
        in_specs=[
            pl.BlockSpec((_BLK, 32), lambda i: (i, 0)),
            pl.BlockSpec((_BLK, 8), lambda i: (i, 0)),
            pl.BlockSpec((8, 32), lambda i: (0, 0)),
            pl.BlockSpec((1, 32), lambda i: (0, 0)),
        ],
        out_specs=[
            pl.BlockSpec((_BLK, 32), lambda i: (i, 0)),
            pl.BlockSpec((8, 128), lambda i: (0, 0)),
        ],
        out_shape=[
            jax.ShapeDtypeStruct((_EP, 32), jnp.float32),
            jax.ShapeDtypeStruct((8, 128), jnp.float32),
        ],
    )(us, pf, w1p, b1)


def _b2_body(h_ref, st_in, g_ref, be_ref, w_ref, b_ref, h2_ref, st_ref):
    i = pl.program_id(0)

    @pl.when(i == 0)
    def _():
        st_ref[...] = jnp.zeros_like(st_ref)

    mu = st_in[0:1, 0:32] / _E
    var = st_in[1:2, 0:32] / _E - mu * mu
    rstd = lax.rsqrt(var + _EPS)
    a = (h_ref[...] - mu) * rstd * g_ref[...] + be_ref[...]
    a = jnp.maximum(a, 0.0)
    h2 = jnp.dot(a, w_ref[...], preferred_element_type=jnp.float32) + b_ref[...]
    h2_ref[...] = h2
    gid = i * _BLK + lax.broadcasted_iota(jnp.int32, (_BLK, 1), 0)
    m = (gid < _E).astype(jnp.float32)
    hm = h2 * m
    st_ref[0:1, 0:32] = st_ref[0:1, 0:32] + jnp.sum(hm, axis=0, keepdims=True)
    st_ref[1:2, 0:32] = st_ref[1:2, 0:32] + jnp.sum(hm * h2, axis=0, keepdims=True)


def _b2(h1, st1, g1, be1, w2, b2):
    return pl.pallas_call(
        _b2_body,
        grid=(_NBLK,),
        in_specs=[
            pl.BlockSpec((_BLK, 32), lambda i: (i, 0)),
            pl.BlockSpec((8, 128), lambda i: (0, 0)),
            pl.BlockSpec((1, 32), lambda i: (0, 0)),
            pl.BlockSpec((1, 32), lambda i: (0, 0)),
            pl.BlockSpec((32, 32), lambda i: (0, 0)),
            pl.BlockSpec((1, 32), lambda i: (0, 0)),
        ],
        out_specs=[
            pl.BlockSpec((_BLK, 32), lambda i: (i, 0)),
            pl.BlockSpec((8, 128), lambda i: (0, 0)),
        ],
        out_shape=[
            jax.ShapeDtypeStruct((_EP, 32), jnp.float32),
            jax.ShapeDtypeStruct((8, 128), jnp.float32),
        ],
    )(h1, st1, g1, be1, w2, b2)


def _b3_body(h_ref, st_in, g_ref, be_ref, w_ref, b_ref, h3_ref):
    mu = st_in[0:1, 0:32] / _E
    var = st_in[1:2, 0:32] / _E - mu * mu
    rstd = lax.rsqrt(var + _EPS)
    a = (h_ref[...] - mu) * rstd * g_ref[...] + be_ref[...]
    a = jnp.maximum(a, 0.0)
    h3_ref[...] = jnp.dot(a, w_ref[...],
                          preferred_element_type=jnp.float32) + b_ref[...]


def _b3(h2, st2, g2, be2, w3, b3):
    return pl.pallas_call(
        _b3_body,
        grid=(_NBLK,),
        in_specs=[
            pl.BlockSpec((_BLK, 32), lambda i: (i, 0)),
            pl.BlockSpec((8, 128), lambda i: (0, 0)),
            pl.BlockSpec((1, 32), lambda i: (0, 0)),
            pl.BlockSpec((1, 32), lambda i: (0, 0)),
            pl.BlockSpec((32, 128), lambda i: (0, 0)),
            pl.BlockSpec((1, 128), lambda i: (0, 0)),
        ],
        out_specs=pl.BlockSpec((_BLK, 128), lambda i: (i, 0)),
        out_shape=jax.ShapeDtypeStruct((_EP, 128), jnp.float32),
    )(h2, st2, g2, be2, w3, b3)


@functools.lru_cache(maxsize=None)
def _build_segmax_sc():
    mesh = plsc.VectorSubcoreMesh(core_axis_name="c", subcore_axis_name="s")

    @functools.partial(
        pl.kernel,
        mesh=mesh,
        out_type=jax.ShapeDtypeStruct((_NP, 128), jnp.float32),
        scratch_types=[
            pltpu.VMEM((_ACC_R, 128), jnp.float32),
            pltpu.VMEM((_CAP,), jnp.int32),
            pltpu.VMEM((_CAP,), jnp.int32),
            pltpu.VMEM((_BLK,), jnp.int32),
            pltpu.VMEM((_CH, 128), jnp.float32),
            pltpu.SemaphoreType.DMA,
        ],
        compiler_params=_sc_params,
    )
    def _segmax_sc(h3_hbm, dst_hbm, out_hbm, acc, dl, el, dc, rb, sem):
        wid = lax.axis_index("s") * 2 + lax.axis_index("c")
        lo = wid * _ROWS_W
        neg = jnp.full((16,), _NEG, jnp.float32)

        @pl.loop(0, _ACC_R)
        def _init(r):
            for cc in range(8):
                acc[r, pl.ds(cc * 16, 16)] = neg

        def _scan_chunk(t, n):
            pltpu.sync_copy(dst_hbm.at[pl.ds(t * _BLK, _BLK)], dc)

            def body(v, n):
                d16 = dc[pl.ds(v * 16, 16)]
                m = (d16 >= lo) & (d16 < lo + _ROWS_W)
                cnt = jnp.sum(jnp.where(m, 1, 0))

                @pl.when(cnt > 0)
                def _():
                    ev = t * _BLK + v * 16 + lax.iota(jnp.int32, 16)
                    plsc.store_compressed(dl.at[pl.ds(n, 16)], d16, mask=m)
                    plsc.store_compressed(el.at[pl.ds(n, 16)], ev, mask=m)

                return n + cnt

            return lax.fori_loop(0, _BLK // 16, body, n)

        n = lax.fori_loop(0, _NBLK, _scan_chunk, 0)

        dummy_d = jnp.full((16,), lo + _ROWS_W, jnp.int32)
        dummy_e = jnp.zeros((16,), jnp.int32)
        tail_idx = n + lax.iota(jnp.int32, 16)
        okm = tail_idx < _CAP
        plsc.store_scatter(dl, [jnp.where(okm, tail_idx, _CAP - 1)],
                           dummy_d, mask=okm)
        plsc.store_scatter(el, [jnp.where(okm, tail_idx, _CAP - 1)],
                           dummy_e, mask=okm)
        npad = (n + _CH - 1) // _CH * _CH

        def _padv(q, _):
            dl[pl.ds(q * 16, 16)] = dummy_d
            el[pl.ds(q * 16, 16)] = dummy_e
            return _

        lax.fori_loop(n // 16 + 1, npad // 16, _padv, 0)

        def _batch(b, _):
            h = pltpu.async_copy(h3_hbm.at[el.at[pl.ds(b * _CH, _CH)]],
                                 rb, sem)
            h.wait()

            def grp(gi, _):
                dv16 = dl[pl.ds(b * _CH + gi * 16, 16)] - lo
                for j in range(16):
                    l = dv16[j]
                    for r in range(8):
                        a = acc[l, pl.ds(r * 16, 16)]
                        v = rb[gi * 16 + j, pl.ds(r * 16, 16)]
                        acc[l, pl.ds(r * 16, 16)] = jnp.maximum(a, v)
                return _

            return lax.fori_loop(0, _CH // 16, grp, 0)

        lax.fori_loop(0, npad // _CH, _batch, 0)

        pltpu.sync_copy(acc.at[pl.ds(0, _ROWS_W)],
                        out_hbm.at[pl.ds(lo, _ROWS_W)])

    return _segmax_sc


def _sm_body(o_ref, r_ref):
    x = o_ref[...]
    rid = lax.broadcasted_iota(jnp.int32, (_NP, 1), 0)
    m = rid < _N
    xm = jnp.where(m, x, _NEG)
    cmax = jnp.max(xm, axis=0, keepdims=True)
    e = jnp.where(m, jnp.exp(xm - cmax), 0.0)
    s = jnp.sum(e, axis=0, keepdims=True)
    r_ref[...] = (e / s)[0:_N, :]


def _softmax(outp):
    return pl.pallas_call(
        _sm_body,
        out_shape=jax.ShapeDtypeStruct((_N, 128), jnp.float32),
    )(outp)


def kernel(x, pos, norm, edge_index, W1, b1, g1, be1, W2, b2, g2, be2, W3, b3):
    ei = edge_index.astype(jnp.int32)
    loops = jnp.arange(_N, dtype=jnp.int32)
    src = jnp.concatenate([ei[0], loops,
                           jnp.zeros((_EP - _E,), jnp.int32)])
    dst = jnp.concatenate([ei[1], loops,
                           jnp.full((_EP - _E,), _NP - 1, jnp.int32)])
    g = jnp.zeros((_NP, 8), jnp.float32)
    g = g.at[:_N, 0:3].set(pos).at[:_N, 3:6].set(norm).reshape(_NP * 8)
    w1x = W1[:128]
    w1p = jnp.zeros((8, 32), jnp.float32).at[0:4].set(W1[128:132])

    u = _node_proj(x, w1x)
    comp, us = _build_gather_sc()(g, u, src, dst)
    ppf_t = _ppf(comp)
    ppf_row = ppf_t.reshape(8, _EP).T
    h1, st1 = _b1(us, ppf_row, w1p, b1[None, :])
    h2, st2 = _b2(h1, st1, g1[None, :], be1[None, :], W2, b2[None, :])
    h3 = _b3(h2, st2, g2[None, :], be2[None, :], W3, b3[None, :])
    outp = _build_segmax_sc()(h3, dst)
    return _softmax(outp)


# speedup vs baseline: 3.9366x; 1.0213x over previous
import dataclasses
import functools

import jax
import jax.numpy as jnp
from jax import lax
from jax.experimental import pallas as pl
from jax.experimental.pallas import tpu as pltpu
from jax.experimental.pallas import tpu_sc as plsc

_N = 10000
_NP = 10240
_E = 330000
_BLK = 2048
_NBLK = 162
_EP = _BLK * _NBLK
_NW = 32
_EW = _EP // _NW
_CH = 128
_NCH = _EW // _CH
_TR = _EP // 128
_ROWS_W = _NP // _NW
_ACC_R = _ROWS_W + 8
_CAP = 16384
_EPS = 1e-5
_NEG = -3.0e38

_sc_params = pltpu.CompilerParams()
if "needs_layout_passes" in pltpu.CompilerParams.__dataclass_fields__:
    _sc_params = dataclasses.replace(_sc_params, needs_layout_passes=False)
if "use_tc_tiling_on_sc" in pltpu.CompilerParams.__dataclass_fields__:
    _sc_params = dataclasses.replace(_sc_params, use_tc_tiling_on_sc=False)


def _u_body(x_ref, w_ref, o_ref):
    o_ref[...] = jnp.dot(x_ref[...], w_ref[...],
                         preferred_element_type=jnp.float32)


def _node_proj(x, w1x):
    return pl.pallas_call(
        _u_body,
        out_shape=jax.ShapeDtypeStruct((_N, 32), jnp.float32),
    )(x, w1x)


@functools.lru_cache(maxsize=None)
def _build_gather_sc():
    mesh = plsc.VectorSubcoreMesh(core_axis_name="c", subcore_axis_name="s")

    @functools.partial(
        pl.kernel,
        mesh=mesh,
        out_type=[
            jax.ShapeDtypeStruct((12, _TR, 128), jnp.float32),
            jax.ShapeDtypeStruct((_EP, 32), jnp.float32),
        ],
        scratch_types=[
            pltpu.VMEM((_NP * 8,), jnp.float32),
            pltpu.VMEM((_CH,), jnp.int32),
            pltpu.VMEM((_CH,), jnp.int32),
            pltpu.VMEM((12, 1, 128), jnp.float32),
            pltpu.VMEM((_CH, 32), jnp.float32),
            pltpu.SemaphoreType.DMA,
        ],
        compiler_params=_sc_params,
    )
    def _gather_sc(g_hbm, u_hbm, src_hbm, dst_hbm, comp_hbm, us_hbm,
                   gv, sv, dv, cb, ub, sem):
        wid = lax.axis_index("s") * 2 + lax.axis_index("c")
        base = wid * _EW
        pltpu.sync_copy(g_hbm, gv)

        @pl.loop(0, _NCH)
        def _chunk(t):
            eb = base + t * _CH
            pltpu.sync_copy(src_hbm.at[pl.ds(eb, _CH)], sv)
            pltpu.sync_copy(dst_hbm.at[pl.ds(eb, _CH)], dv)
            h = pltpu.async_copy(u_hbm.at[sv], ub, sem)

            @pl.loop(0, _CH // 16)
            def _vec(v):
                si = sv[pl.ds(v * 16, 16)] * 8
                di = dv[pl.ds(v * 16, 16)] * 8
                for k in range(6):
                    cb[k, 0, pl.ds(v * 16, 16)] = plsc.load_gather(gv, [si + k])
                    cb[6 + k, 0, pl.ds(v * 16, 16)] = plsc.load_gather(
                        gv, [di + k])

            pltpu.sync_copy(cb, comp_hbm.at[:, pl.ds(eb // 128, 1), :])
            h.wait()
            pltpu.sync_copy(ub, us_hbm.at[pl.ds(eb, _CH)])

    return _gather_sc


def _safenorm(x, y, z):
    sq = x * x + y * y + z * z
    return jnp.where(sq > 0, jnp.sqrt(jnp.where(sq > 0, sq, 1.0)), 0.0)


def _angle(ax, ay, az, bx, by, bz):
    cx = ay * bz - az * by
    cy = az * bx - ax * bz
    cz = ax * by - ay * bx
    dot = ax * bx + ay * by + az * bz
    dot = jnp.where(dot == 0.0, 0.0, dot)
    return jnp.arctan2(_safenorm(cx, cy, cz), dot)


def _ppf_body(c_ref, o_ref):
    c = c_ref[...]
    pjx, pjy, pjz = c[0], c[1], c[2]
    njx, njy, njz = c[3], c[4], c[5]
    pix, piy, piz = c[6], c[7], c[8]
    nix, niy, niz = c[9], c[10], c[11]
    dx, dy, dz = pjx - pix, pjy - piy, pjz - piz
    f1 = _safenorm(dx, dy, dz)
    f2 = _angle(nix, niy, niz, dx, dy, dz)
    f3 = _angle(njx, njy, njz, dx, dy, dz)
    f4 = _angle(nix, niy, niz, njx, njy, njz)
    z = jnp.zeros_like(f1)
    o_ref[...] = jnp.stack([f1, f2, f3, f4, z, z, z, z], axis=0)


def _ppf(comp):
    return pl.pallas_call(
        _ppf_body,
        grid=(_TR // 16,),
        in_specs=[pl.BlockSpec((12, 16, 128), lambda i: (0, i, 0))],
        out_specs=pl.BlockSpec((8, 16, 128), lambda i: (0, i, 0)),
        out_shape=jax.ShapeDtypeStruct((8, _TR, 128), jnp.float32),
    )(comp)


def _b1_body(us_ref, pf_ref, w_ref, b_ref, h_ref, st_ref):
    i = pl.program_id(0)

    @pl.when(i == 0)
    def _():
        st_ref[...] = jnp.zeros_like(st_ref)

    h = us_ref[...] + jnp.dot(w_ref[...], pf_ref[...],
                              preferred_element_type=jnp.float32) + b_ref[...]
    h_ref[...] = h
    gid = i * _BLK + lax.broadcasted_iota(jnp.int32, (1, _BLK), 1)
    m = (gid < _E).astype(jnp.float32)
    hm = h * m
    st_ref[0:32, 0:1] = st_ref[0:32, 0:1] + jnp.sum(hm, axis=1, keepdims=True)
    st_ref[0:32, 1:2] = st_ref[0:32, 1:2] + jnp.sum(hm * h, axis=1,
                                                    keepdims=True)


def _b1(us_t, pf8, w1pt, b1c):
    return pl.pallas_call(
        _b1_body,
        grid=(_NBLK,),
        in_specs=[
            pl.BlockSpec((32, _BLK), lambda i: (0, i)),
            pl.BlockSpec((8, _BLK), lambda i: (0, i)),
            pl.BlockSpec((32, 8), lambda i: (0, 0)),
            pl.BlockSpec((32, 1), lambda i: (0, 0)),
        ],
        out_specs=[
            pl.BlockSpec((32, _BLK), lambda i: (0, i)),
            pl.BlockSpec((32, 128), lambda i: (0, 0)),
        ],
        out_shape=[
            jax.ShapeDtypeStruct((32, _EP), jnp.float32),
            jax.ShapeDtypeStruct((32, 128), jnp.float32),
        ],
    )(us_t, pf8, w1pt, b1c)


def _b2_body(h_ref, st_in, g_ref, be_ref, w_ref, b_ref, h2_ref, st_ref):
    i = pl.program_id(0)

    @pl.when(i == 0)
    def _():
        st_ref[...] = jnp.zeros_like(st_ref)

    mu = st_in[0:32, 0:1] / _E
    var = st_in[0:32, 1:2] / _E - mu * mu
    rstd = lax.rsqrt(var + _EPS)
    a = (h_ref[...] - mu) * rstd * g_ref[...] + be_ref[...]
    a = jnp.maximum(a, 0.0)
    h2 = jnp.dot(w_ref[...], a, preferred_element_type=jnp.float32) + b_ref[...]
    h2_ref[...] = h2
    gid = i * _BLK + lax.broadcasted_iota(jnp.int32, (1, _BLK), 1)
    m = (gid < _E).astype(jnp.float32)
    hm = h2 * m
    st_ref[0:32, 0:1] = st_ref[0:32, 0:1] + jnp.sum(hm, axis=1, keepdims=True)
    st_ref[0:32, 1:2] = st_ref[0:32, 1:2] + jnp.sum(hm * h2, axis=1,
                                                    keepdims=True)


def _b2(h1_t, st1, g1c, be1c, w2t, b2c):
    return pl.pallas_call(
        _b2_body,
        grid=(_NBLK,),
        in_specs=[
            pl.BlockSpec((32, _BLK), lambda i: (0, i)),
            pl.BlockSpec((32, 128), lambda i: (0, 0)),
            pl.BlockSpec((32, 1), lambda i: (0, 0)),
            pl.BlockSpec((32, 1), lambda i: (0, 0)),
            pl.BlockSpec((32, 32), lambda i: (0, 0)),
            pl.BlockSpec((32, 1), lambda i: (0, 0)),
        ],
        out_specs=[
            pl.BlockSpec((32, _BLK), lambda i: (0, i)),
            pl.BlockSpec((32, 128), lambda i: (0, 0)),
        ],
        out_shape=[
            jax.ShapeDtypeStruct((32, _EP), jnp.float32),
            jax.ShapeDtypeStruct((32, 128), jnp.float32),
        ],
    )(h1_t, st1, g1c, be1c, w2t, b2c)


def _b3_body(h_ref, st_in, g_ref, be_ref, w_ref, b_ref, h3_ref):
    mu = st_in[0:32, 0:1] / _E
    var = st_in[0:32, 1:2] / _E - mu * mu
    rstd = lax.rsqrt(var + _EPS)
    a = (h_ref[...] - mu) * rstd * g_ref[...] + be_ref[...]
    a = jnp.maximum(a, 0.0)
    h3 = lax.dot_general(a, w_ref[...], (((0,), (0,)), ((), ())),
                         preferred_element_type=jnp.float32)
    h3_ref[...] = h3 + b_ref[...]


def _b3(h2_t, st2, g2c, be2c, w3, b3r):
    return pl.pallas_call(
        _b3_body,
        grid=(_NBLK,),
        in_specs=[
            pl.BlockSpec((32, _BLK), lambda i: (0, i)),
            pl.BlockSpec((32, 128), lambda i: (0, 0)),
            pl.BlockSpec((32, 1), lambda i: (0, 0)),
            pl.BlockSpec((32, 1), lambda i: (0, 0)),
            pl.BlockSpec((32, 128), lambda i: (0, 0)),
            pl.BlockSpec((1, 128), lambda i: (0, 0)),
        ],
        out_specs=pl.BlockSpec((_BLK, 128), lambda i: (i, 0)),
        out_shape=jax.ShapeDtypeStruct((_EP, 128), jnp.float32),
    )(h2_t, st2, g2c, be2c, w3, b3r)


@functools.lru_cache(maxsize=None)
def _build_segmax_sc():
    mesh = plsc.VectorSubcoreMesh(core_axis_name="c", subcore_axis_name="s")

    @functools.partial(
        pl.kernel,
        mesh=mesh,
        out_type=jax.ShapeDtypeStruct((_NP, 128), jnp.float32),
        scratch_types=[
            pltpu.VMEM((_ACC_R, 128), jnp.float32),
            pltpu.VMEM((_CAP,), jnp.int32),
            pltpu.VMEM((_CAP,), jnp.int32),
            pltpu.VMEM((_BLK,), jnp.int32),
            pltpu.VMEM((_CH, 128), jnp.float32),
            pltpu.VMEM((_CH, 128), jnp.float32),
            pltpu.SemaphoreType.DMA,
            pltpu.SemaphoreType.DMA,
        ],
        compiler_params=_sc_params,
    )
    def _segmax_sc(h3_hbm, dst_hbm, out_hbm, acc, dl, el, dc, rb0, rb1,
                   sem0, sem1):
        wid = lax.axis_index("s") * 2 + lax.axis_index("c")
        lo = wid * _ROWS_W
        neg = jnp.full((16,), _NEG, jnp.float32)

        @pl.loop(0, _ACC_R)
        def _init(r):
            for cc in range(8):
                acc[r, pl.ds(cc * 16, 16)] = neg

        limit = jnp.uint32(_ROWS_W)

        def _scan_chunk(t, n):
            pltpu.sync_copy(dst_hbm.at[pl.ds(t * _BLK, _BLK)], dc)

            def body(v, n):
                d16 = dc[pl.ds(v * 16, 16)]
                m = plsc.bitcast(d16 - lo, jnp.uint32) < limit
                cnt = plsc.all_reduce_population_count(m)[0]

                @pl.when(cnt > 0)
                def _():
                    ev = t * _BLK + v * 16 + lax.iota(jnp.int32, 16)
                    plsc.store_compressed(dl.at[pl.ds(n, 16)], d16, mask=m)
                    plsc.store_compressed(el.at[pl.ds(n, 16)], ev, mask=m)

                return n + cnt

            return lax.fori_loop(0, _BLK // 16, body, n)

        n = lax.fori_loop(0, _NBLK, _scan_chunk, 0)

        dummy_d = jnp.full((16,), lo + _ROWS_W, jnp.int32)
        dummy_e = jnp.zeros((16,), jnp.int32)
        tail_idx = n + lax.iota(jnp.int32, 16)
        okm = tail_idx < _CAP
        plsc.store_scatter(dl, [jnp.where(okm, tail_idx, _CAP - 1)],
                           dummy_d, mask=okm)
        plsc.store_scatter(el, [jnp.where(okm, tail_idx, _CAP - 1)],
                           dummy_e, mask=okm)
        npad = (n + 2 * _CH - 1) // (2 * _CH) * (2 * _CH)

        def _padv(q, _):
            dl[pl.ds(q * 16, 16)] = dummy_d
            el[pl.ds(q * 16, 16)] = dummy_e
            return _

        lax.fori_loop(n // 16 + 1, npad // 16, _padv, 0)

        nb = npad // _CH

        def _issue(b, rb, sem):
            bc = jnp.minimum(b, nb - 1)
            return pltpu.async_copy(
                h3_hbm.at[el.at[pl.ds(bc * _CH, _CH)]], rb, sem)

        def _wait(rb, sem):
            pltpu.make_async_copy(
                h3_hbm.at[el.at[pl.ds(0, _CH)]], rb, sem).wait()

        def _rmw(b, rb):
            def grp(gi, _):
                dv16 = dl[pl.ds(b * _CH + gi * 16, 16)] - lo
                for j in range(16):
                    l = dv16[j]
                    for r in range(8):
                        a = acc[l, pl.ds(r * 16, 16)]
                        v = rb[gi * 16 + j, pl.ds(r * 16, 16)]
                        acc[l, pl.ds(r * 16, 16)] = jnp.maximum(a, v)
                return _

            lax.fori_loop(0, _CH // 16, grp, 0)

        _issue(0, rb0, sem0)
        _issue(1, rb1, sem1)

        def _pair(p, _):
            _wait(rb0, sem0)
            _rmw(2 * p, rb0)
            _issue(2 * p + 2, rb0, sem0)
            _wait(rb1, sem1)
            _rmw(2 * p + 1, rb1)
            _issue(2 * p + 3, rb1, sem1)
            return _

        lax.fori_loop(0, nb // 2, _pair, 0)
        _wait(rb0, sem0)
        _wait(rb1, sem1)

        pltpu.sync_copy(acc.at[pl.ds(0, _ROWS_W)],
                        out_hbm.at[pl.ds(lo, _ROWS_W)])

    return _segmax_sc


def _sm_body(o_ref, r_ref):
    x = o_ref[...]
    rid = lax.broadcasted_iota(jnp.int32, (_NP, 1), 0)
    m = rid < _N
    xm = jnp.where(m, x, _NEG)
    cmax = jnp.max(xm, axis=0, keepdims=True)
    e = jnp.where(m, jnp.exp(xm - cmax), 0.0)
    s = jnp.sum(e, axis=0, keepdims=True)
    r_ref[...] = (e / s)[0:_N, :]


def _softmax(outp):
    return pl.pallas_call(
        _sm_body,
        out_shape=jax.ShapeDtypeStruct((_N, 128), jnp.float32),
    )(outp)


def kernel(x, pos, norm, edge_index, W1, b1, g1, be1, W2, b2, g2, be2, W3, b3):
    ei = edge_index.astype(jnp.int32)
    loops = jnp.arange(_N, dtype=jnp.int32)
    src = jnp.concatenate([ei[0], loops,
                           jnp.zeros((_EP - _E,), jnp.int32)])
    dst = jnp.concatenate([ei[1], loops,
                           jnp.full((_EP - _E,), _NP - 1, jnp.int32)])
    g = jnp.zeros((_NP, 8), jnp.float32)
    g = g.at[:_N, 0:3].set(pos).at[:_N, 3:6].set(norm).reshape(_NP * 8)
    w1x = W1[:128]
    w1pt = jnp.zeros((32, 8), jnp.float32).at[:, 0:4].set(W1[128:132].T)

    u = _node_proj(x, w1x)
    comp, us = _build_gather_sc()(g, u, src, dst)
    ppf8 = _ppf(comp).reshape(8, _EP)
    us_t = us.T
    h1_t, st1 = _b1(us_t, ppf8, w1pt, b1[:, None])
    h2_t, st2 = _b2(h1_t, st1, g1[:, None], be1[:, None], W2.T, b2[:, None])
    h3 = _b3(h2_t, st2, g2[:, None], be2[:, None], W3, b3[None, :])
    outp = _build_segmax_sc()(h3, dst)
    return _softmax(outp)


# speedup vs baseline: 4.7372x; 1.2034x over previous
import dataclasses
import functools

import jax
import jax.numpy as jnp
from jax import lax
from jax.experimental import pallas as pl
from jax.experimental.pallas import tpu as pltpu
from jax.experimental.pallas import tpu_sc as plsc

_N = 10000
_NP = 10240
_E = 330000
_BLK = 2048
_NBLK = 162
_EP = _BLK * _NBLK
_NW = 32
_EW = _EP // _NW
_CH = 128
_NCH = _EW // _CH
_TR = _EP // 128
_ROWS_W = _NP // _NW
_ACC_R = _ROWS_W + 8
_CAP = 16384
_EPS = 1e-5
_NEG = -3.0e38

_sc_params = pltpu.CompilerParams()
if "needs_layout_passes" in pltpu.CompilerParams.__dataclass_fields__:
    _sc_params = dataclasses.replace(_sc_params, needs_layout_passes=False)
if "use_tc_tiling_on_sc" in pltpu.CompilerParams.__dataclass_fields__:
    _sc_params = dataclasses.replace(_sc_params, use_tc_tiling_on_sc=False)


def _u_body(x_ref, w_ref, o_ref):
    o_ref[...] = jnp.dot(x_ref[...], w_ref[...],
                         preferred_element_type=jnp.float32)


def _node_proj(x, w1x):
    return pl.pallas_call(
        _u_body,
        out_shape=jax.ShapeDtypeStruct((_N, 32), jnp.float32),
    )(x, w1x)


@functools.lru_cache(maxsize=None)
def _build_gather_sc():
    mesh = plsc.VectorSubcoreMesh(core_axis_name="c", subcore_axis_name="s")

    @functools.partial(
        pl.kernel,
        mesh=mesh,
        out_type=[
            jax.ShapeDtypeStruct((12, _TR, 128), jnp.float32),
            jax.ShapeDtypeStruct((_EP, 32), jnp.float32),
        ],
        scratch_types=[
            pltpu.VMEM((_NP * 8,), jnp.float32),
            pltpu.VMEM((_CH,), jnp.int32),
            pltpu.VMEM((_CH,), jnp.int32),
            pltpu.VMEM((12, 1, 128), jnp.float32),
            pltpu.VMEM((_CH, 32), jnp.float32),
            pltpu.SemaphoreType.DMA,
        ],
        compiler_params=_sc_params,
    )
    def _gather_sc(g_hbm, u_hbm, src_hbm, dst_hbm, comp_hbm, us_hbm,
                   gv, sv, dv, cb, ub, sem):
        wid = lax.axis_index("s") * 2 + lax.axis_index("c")
        base = wid * _EW
        pltpu.sync_copy(g_hbm, gv)

        @pl.loop(0, _NCH)
        def _chunk(t):
            eb = base + t * _CH
            pltpu.sync_copy(src_hbm.at[pl.ds(eb, _CH)], sv)
            pltpu.sync_copy(dst_hbm.at[pl.ds(eb, _CH)], dv)
            h = pltpu.async_copy(u_hbm.at[sv], ub, sem)

            @pl.loop(0, _CH // 16)
            def _vec(v):
                si = sv[pl.ds(v * 16, 16)] * 8
                di = dv[pl.ds(v * 16, 16)] * 8
                for k in range(6):
                    cb[k, 0, pl.ds(v * 16, 16)] = plsc.load_gather(gv, [si + k])
                    cb[6 + k, 0, pl.ds(v * 16, 16)] = plsc.load_gather(
                        gv, [di + k])

            pltpu.sync_copy(cb, comp_hbm.at[:, pl.ds(eb // 128, 1), :])
            h.wait()
            pltpu.sync_copy(ub, us_hbm.at[pl.ds(eb, _CH)])

    return _gather_sc


def _safenorm(x, y, z):
    sq = x * x + y * y + z * z
    return jnp.where(sq > 0, jnp.sqrt(jnp.where(sq > 0, sq, 1.0)), 0.0)


def _angle(ax, ay, az, bx, by, bz):
    cx = ay * bz - az * by
    cy = az * bx - ax * bz
    cz = ax * by - ay * bx
    dot = ax * bx + ay * by + az * bz
    dot = jnp.where(dot == 0.0, 0.0, dot)
    return jnp.arctan2(_safenorm(cx, cy, cz), dot)


def _ppf_body(c_ref, o_ref):
    c = c_ref[...]
    pjx, pjy, pjz = c[0], c[1], c[2]
    njx, njy, njz = c[3], c[4], c[5]
    pix, piy, piz = c[6], c[7], c[8]
    nix, niy, niz = c[9], c[10], c[11]
    dx, dy, dz = pjx - pix, pjy - piy, pjz - piz
    f1 = _safenorm(dx, dy, dz)
    f2 = _angle(nix, niy, niz, dx, dy, dz)
    f3 = _angle(njx, njy, njz, dx, dy, dz)
    f4 = _angle(nix, niy, niz, njx, njy, njz)
    z = jnp.zeros_like(f1)
    o_ref[...] = jnp.stack([f1, f2, f3, f4, z, z, z, z], axis=0)


def _ppf(comp):
    return pl.pallas_call(
        _ppf_body,
        grid=(_TR // 16,),
        in_specs=[pl.BlockSpec((12, 16, 128), lambda i: (0, i, 0))],
        out_specs=pl.BlockSpec((8, 16, 128), lambda i: (0, i, 0)),
        out_shape=jax.ShapeDtypeStruct((8, _TR, 128), jnp.float32),
    )(comp)


def _b1_body(us_ref, pf_ref, w_ref, b_ref, h_ref, st_ref):
    i = pl.program_id(0)

    @pl.when(i == 0)
    def _():
        st_ref[...] = jnp.zeros_like(st_ref)

    h = us_ref[...] + jnp.dot(w_ref[...], pf_ref[...],
                              preferred_element_type=jnp.float32) + b_ref[...]
    h_ref[...] = h
    gid = i * _BLK + lax.broadcasted_iota(jnp.int32, (1, _BLK), 1)
    m = (gid < _E).astype(jnp.float32)
    hm = h * m
    st_ref[0:32, 0:1] = st_ref[0:32, 0:1] + jnp.sum(hm, axis=1, keepdims=True)
    st_ref[0:32, 1:2] = st_ref[0:32, 1:2] + jnp.sum(hm * h, axis=1,
                                                    keepdims=True)


def _b1(us_t, pf8, w1pt, b1c):
    return pl.pallas_call(
        _b1_body,
        grid=(_NBLK,),
        in_specs=[
            pl.BlockSpec((32, _BLK), lambda i: (0, i)),
            pl.BlockSpec((8, _BLK), lambda i: (0, i)),
            pl.BlockSpec((32, 8), lambda i: (0, 0)),
            pl.BlockSpec((32, 1), lambda i: (0, 0)),
        ],
        out_specs=[
            pl.BlockSpec((32, _BLK), lambda i: (0, i)),
            pl.BlockSpec((32, 128), lambda i: (0, 0)),
        ],
        out_shape=[
            jax.ShapeDtypeStruct((32, _EP), jnp.float32),
            jax.ShapeDtypeStruct((32, 128), jnp.float32),
        ],
    )(us_t, pf8, w1pt, b1c)


def _b2_body(h_ref, st_in, g_ref, be_ref, w_ref, b_ref, h2_ref, st_ref):
    i = pl.program_id(0)

    @pl.when(i == 0)
    def _():
        st_ref[...] = jnp.zeros_like(st_ref)

    mu = st_in[0:32, 0:1] / _E
    var = st_in[0:32, 1:2] / _E - mu * mu
    rstd = lax.rsqrt(var + _EPS)
    a = (h_ref[...] - mu) * rstd * g_ref[...] + be_ref[...]
    a = jnp.maximum(a, 0.0)
    h2 = jnp.dot(w_ref[...], a, preferred_element_type=jnp.float32) + b_ref[...]
    h2_ref[...] = h2
    gid = i * _BLK + lax.broadcasted_iota(jnp.int32, (1, _BLK), 1)
    m = (gid < _E).astype(jnp.float32)
    hm = h2 * m
    st_ref[0:32, 0:1] = st_ref[0:32, 0:1] + jnp.sum(hm, axis=1, keepdims=True)
    st_ref[0:32, 1:2] = st_ref[0:32, 1:2] + jnp.sum(hm * h2, axis=1,
                                                    keepdims=True)


def _b2(h1_t, st1, g1c, be1c, w2t, b2c):
    return pl.pallas_call(
        _b2_body,
        grid=(_NBLK,),
        in_specs=[
            pl.BlockSpec((32, _BLK), lambda i: (0, i)),
            pl.BlockSpec((32, 128), lambda i: (0, 0)),
            pl.BlockSpec((32, 1), lambda i: (0, 0)),
            pl.BlockSpec((32, 1), lambda i: (0, 0)),
            pl.BlockSpec((32, 32), lambda i: (0, 0)),
            pl.BlockSpec((32, 1), lambda i: (0, 0)),
        ],
        out_specs=[
            pl.BlockSpec((32, _BLK), lambda i: (0, i)),
            pl.BlockSpec((32, 128), lambda i: (0, 0)),
        ],
        out_shape=[
            jax.ShapeDtypeStruct((32, _EP), jnp.float32),
            jax.ShapeDtypeStruct((32, 128), jnp.float32),
        ],
    )(h1_t, st1, g1c, be1c, w2t, b2c)


def _b3_body(h_ref, st_in, g_ref, be_ref, w_ref, b_ref, h3_ref):
    mu = st_in[0:32, 0:1] / _E
    var = st_in[0:32, 1:2] / _E - mu * mu
    rstd = lax.rsqrt(var + _EPS)
    a = (h_ref[...] - mu) * rstd * g_ref[...] + be_ref[...]
    a = jnp.maximum(a, 0.0)
    h3 = lax.dot_general(a, w_ref[...], (((0,), (0,)), ((), ())),
                         preferred_element_type=jnp.float32)
    h3_ref[...] = h3 + b_ref[...]


def _b3(h2_t, st2, g2c, be2c, w3, b3r):
    return pl.pallas_call(
        _b3_body,
        grid=(_NBLK,),
        in_specs=[
            pl.BlockSpec((32, _BLK), lambda i: (0, i)),
            pl.BlockSpec((32, 128), lambda i: (0, 0)),
            pl.BlockSpec((32, 1), lambda i: (0, 0)),
            pl.BlockSpec((32, 1), lambda i: (0, 0)),
            pl.BlockSpec((32, 128), lambda i: (0, 0)),
            pl.BlockSpec((1, 128), lambda i: (0, 0)),
        ],
        out_specs=pl.BlockSpec((_BLK, 128), lambda i: (i, 0)),
        out_shape=jax.ShapeDtypeStruct((_EP, 128), jnp.float32),
    )(h2_t, st2, g2c, be2c, w3, b3r)


@functools.lru_cache(maxsize=None)
def _build_segmax_sc():
    mesh = plsc.VectorSubcoreMesh(core_axis_name="c", subcore_axis_name="s")

    @functools.partial(
        pl.kernel,
        mesh=mesh,
        out_type=jax.ShapeDtypeStruct((_NP, 128), jnp.float32),
        scratch_types=[
            pltpu.VMEM((_ACC_R, 128), jnp.float32),
            pltpu.VMEM((_CAP,), jnp.int32),
            pltpu.VMEM((_CAP,), jnp.int32),
            pltpu.VMEM((_BLK,), jnp.int32),
            pltpu.VMEM((_BLK,), jnp.int32),
            pltpu.VMEM((_CH, 128), jnp.float32),
            pltpu.VMEM((_CH, 128), jnp.float32),
            pltpu.SemaphoreType.DMA,
            pltpu.SemaphoreType.DMA,
            pltpu.SemaphoreType.DMA,
            pltpu.SemaphoreType.DMA,
        ],
        compiler_params=_sc_params,
    )
    def _segmax_sc(h3_hbm, dst_hbm, out_hbm, acc, dl, el, dc0, dc1, rb0, rb1,
                   sem0, sem1, sem2, sem3):
        wid = lax.axis_index("s") * 2 + lax.axis_index("c")
        lo = wid * _ROWS_W
        neg = jnp.full((16,), _NEG, jnp.float32)

        @pl.loop(0, _ACC_R)
        def _init(r):
            for cc in range(8):
                acc[r, pl.ds(cc * 16, 16)] = neg

        limit = jnp.uint32(_ROWS_W)
        iota16 = lax.iota(jnp.int32, 16)

        def _dissue(t, dc, sem):
            tc = jnp.minimum(t, _NBLK - 1)
            return pltpu.async_copy(dst_hbm.at[pl.ds(tc * _BLK, _BLK)],
                                    dc, sem)

        def _dwait(dc, sem):
            pltpu.make_async_copy(dst_hbm.at[pl.ds(0, _BLK)], dc, sem).wait()

        def _scan_chunk(t, dc, n):
            def body(v, n):
                base = t * _BLK + v * 64
                ds_ = [dc[pl.ds(v * 64 + 16 * k, 16)] for k in range(4)]
                ms = [plsc.bitcast(d - lo, jnp.uint32) < limit for d in ds_]
                cs = [plsc.all_reduce_population_count(m)[0] for m in ms]
                for k in range(4):
                    ev = base + 16 * k + iota16
                    plsc.store_compressed(dl.at[pl.ds(n, 16)], ds_[k],
                                          mask=ms[k])
                    plsc.store_compressed(el.at[pl.ds(n, 16)], ev,
                                          mask=ms[k])
                    n = n + cs[k]
                return n

            return lax.fori_loop(0, _BLK // 64, body, n)

        _dissue_h = _dissue(0, dc0, sem2)
        _dissue_h = _dissue(1, dc1, sem3)

        def _scan_pair(q, n):
            _dwait(dc0, sem2)
            n = _scan_chunk(2 * q, dc0, n)
            _dissue(2 * q + 2, dc0, sem2)
            _dwait(dc1, sem3)
            n = _scan_chunk(2 * q + 1, dc1, n)
            _dissue(2 * q + 3, dc1, sem3)
            return n

        n = lax.fori_loop(0, _NBLK // 2, _scan_pair, 0)
        _dwait(dc0, sem2)
        _dwait(dc1, sem3)

        dummy_d = jnp.full((16,), lo + _ROWS_W, jnp.int32)
        dummy_e = jnp.zeros((16,), jnp.int32)
        tail_idx = n + lax.iota(jnp.int32, 16)
        okm = tail_idx < _CAP
        plsc.store_scatter(dl, [jnp.where(okm, tail_idx, _CAP - 1)],
                           dummy_d, mask=okm)
        plsc.store_scatter(el, [jnp.where(okm, tail_idx, _CAP - 1)],
                           dummy_e, mask=okm)
        npad = (n + 2 * _CH - 1) // (2 * _CH) * (2 * _CH)

        def _padv(q, _):
            dl[pl.ds(q * 16, 16)] = dummy_d
            el[pl.ds(q * 16, 16)] = dummy_e
            return _

        lax.fori_loop(n // 16 + 1, npad // 16, _padv, 0)

        nb = npad // _CH

        def _issue(b, rb, sem):
            bc = jnp.minimum(b, nb - 1)
            return pltpu.async_copy(
                h3_hbm.at[el.at[pl.ds(bc * _CH, _CH)]], rb, sem)

        def _wait(rb, sem):
            pltpu.make_async_copy(
                h3_hbm.at[el.at[pl.ds(0, _CH)]], rb, sem).wait()

        def _rmw(b, rb):
            def grp(gi, _):
                dv16 = dl[pl.ds(b * _CH + gi * 16, 16)] - lo
                for j in range(16):
                    l = dv16[j]
                    for r in range(8):
                        a = acc[l, pl.ds(r * 16, 16)]
                        v = rb[gi * 16 + j, pl.ds(r * 16, 16)]
                        acc[l, pl.ds(r * 16, 16)] = jnp.maximum(a, v)
                return _

            lax.fori_loop(0, _CH // 16, grp, 0)

        _issue(0, rb0, sem0)
        _issue(1, rb1, sem1)

        def _pair(p, _):
            _wait(rb0, sem0)
            _rmw(2 * p, rb0)
            _issue(2 * p + 2, rb0, sem0)
            _wait(rb1, sem1)
            _rmw(2 * p + 1, rb1)
            _issue(2 * p + 3, rb1, sem1)
            return _

        lax.fori_loop(0, nb // 2, _pair, 0)
        _wait(rb0, sem0)
        _wait(rb1, sem1)

        pltpu.sync_copy(acc.at[pl.ds(0, _ROWS_W)],
                        out_hbm.at[pl.ds(lo, _ROWS_W)])

    return _segmax_sc


def _sm_body(o_ref, r_ref):
    x = o_ref[...]
    rid = lax.broadcasted_iota(jnp.int32, (_NP, 1), 0)
    m = rid < _N
    xm = jnp.where(m, x, _NEG)
    cmax = jnp.max(xm, axis=0, keepdims=True)
    e = jnp.where(m, jnp.exp(xm - cmax), 0.0)
    s = jnp.sum(e, axis=0, keepdims=True)
    r_ref[...] = (e / s)[0:_N, :]


def _softmax(outp):
    return pl.pallas_call(
        _sm_body,
        out_shape=jax.ShapeDtypeStruct((_N, 128), jnp.float32),
    )(outp)


def kernel(x, pos, norm, edge_index, W1, b1, g1, be1, W2, b2, g2, be2, W3, b3):
    ei = edge_index.astype(jnp.int32)
    loops = jnp.arange(_N, dtype=jnp.int32)
    src = jnp.concatenate([ei[0], loops,
                           jnp.zeros((_EP - _E,), jnp.int32)])
    dst = jnp.concatenate([ei[1], loops,
                           jnp.full((_EP - _E,), _NP - 1, jnp.int32)])
    g = jnp.zeros((_NP, 8), jnp.float32)
    g = g.at[:_N, 0:3].set(pos).at[:_N, 3:6].set(norm).reshape(_NP * 8)
    w1x = W1[:128]
    w1pt = jnp.zeros((32, 8), jnp.float32).at[:, 0:4].set(W1[128:132].T)

    u = _node_proj(x, w1x)
    comp, us = _build_gather_sc()(g, u, src, dst)
    ppf8 = _ppf(comp).reshape(8, _EP)
    us_t = us.T
    h1_t, st1 = _b1(us_t, ppf8, w1pt, b1[:, None])
    h2_t, st2 = _b2(h1_t, st1, g1[:, None], be1[:, None], W2.T, b2[:, None])
    h3 = _b3(h2_t, st2, g2[:, None], be2[:, None], W3, b3[None, :])
    outp = _build_segmax_sc()(h3, dst)
    return _softmax(outp)


# speedup vs baseline: 5.0211x; 1.0599x over previous
import dataclasses
import functools

import jax
import jax.numpy as jnp
from jax import lax
from jax.experimental import pallas as pl
from jax.experimental.pallas import tpu as pltpu
from jax.experimental.pallas import tpu_sc as plsc

_N = 10000
_NP = 10240
_E = 330000
_BLK = 2048
_NBLK = 162
_EP = _BLK * _NBLK
_NW = 32
_EW = _EP // _NW
_CH = 128
_NCH = _EW // _CH
_TR = _EP // 128
_ROWS_W = _NP // _NW
_ACC_R = _ROWS_W + 8
_CAP = 16384
_EPS = 1e-5
_NEG = -3.0e38

_sc_params = pltpu.CompilerParams()
if "needs_layout_passes" in pltpu.CompilerParams.__dataclass_fields__:
    _sc_params = dataclasses.replace(_sc_params, needs_layout_passes=False)
if "use_tc_tiling_on_sc" in pltpu.CompilerParams.__dataclass_fields__:
    _sc_params = dataclasses.replace(_sc_params, use_tc_tiling_on_sc=False)


def _u_body(x_ref, w_ref, o_ref):
    o_ref[...] = jnp.dot(x_ref[...], w_ref[...],
                         preferred_element_type=jnp.float32)


def _node_proj(x, w1x):
    return pl.pallas_call(
        _u_body,
        out_shape=jax.ShapeDtypeStruct((_N, 32), jnp.float32),
    )(x, w1x)


@functools.lru_cache(maxsize=None)
def _build_gather_sc():
    mesh = plsc.VectorSubcoreMesh(core_axis_name="c", subcore_axis_name="s")

    @functools.partial(
        pl.kernel,
        mesh=mesh,
        out_type=[
            jax.ShapeDtypeStruct((12, _TR, 128), jnp.float32),
            jax.ShapeDtypeStruct((_EP, 32), jnp.float32),
        ],
        scratch_types=[
            pltpu.VMEM((_NP * 8,), jnp.float32),
            pltpu.VMEM((_CH,), jnp.int32),
            pltpu.VMEM((_CH,), jnp.int32),
            pltpu.VMEM((12, 1, 128), jnp.float32),
            pltpu.VMEM((_CH, 32), jnp.float32),
            pltpu.SemaphoreType.DMA,
        ],
        compiler_params=_sc_params,
    )
    def _gather_sc(g_hbm, u_hbm, src_hbm, dst_hbm, comp_hbm, us_hbm,
                   gv, sv, dv, cb, ub, sem):
        wid = lax.axis_index("s") * 2 + lax.axis_index("c")
        base = wid * _EW
        pltpu.sync_copy(g_hbm, gv)

        @pl.loop(0, _NCH)
        def _chunk(t):
            eb = base + t * _CH
            pltpu.sync_copy(src_hbm.at[pl.ds(eb, _CH)], sv)
            pltpu.sync_copy(dst_hbm.at[pl.ds(eb, _CH)], dv)
            h = pltpu.async_copy(u_hbm.at[sv], ub, sem)

            @pl.loop(0, _CH // 16)
            def _vec(v):
                si = sv[pl.ds(v * 16, 16)] * 8
                di = dv[pl.ds(v * 16, 16)] * 8
                for k in range(6):
                    cb[k, 0, pl.ds(v * 16, 16)] = plsc.load_gather(gv, [si + k])
                    cb[6 + k, 0, pl.ds(v * 16, 16)] = plsc.load_gather(
                        gv, [di + k])

            pltpu.sync_copy(cb, comp_hbm.at[:, pl.ds(eb // 128, 1), :])
            h.wait()
            pltpu.sync_copy(ub, us_hbm.at[pl.ds(eb, _CH)])

    return _gather_sc


def _safenorm(x, y, z):
    sq = x * x + y * y + z * z
    return jnp.where(sq > 0, jnp.sqrt(jnp.where(sq > 0, sq, 1.0)), 0.0)


def _angle(ax, ay, az, bx, by, bz):
    cx = ay * bz - az * by
    cy = az * bx - ax * bz
    cz = ax * by - ay * bx
    dot = ax * bx + ay * by + az * bz
    dot = jnp.where(dot == 0.0, 0.0, dot)
    return jnp.arctan2(_safenorm(cx, cy, cz), dot)


def _ppf_body(c_ref, o_ref):
    c = c_ref[...]
    pjx, pjy, pjz = c[0], c[1], c[2]
    njx, njy, njz = c[3], c[4], c[5]
    pix, piy, piz = c[6], c[7], c[8]
    nix, niy, niz = c[9], c[10], c[11]
    dx, dy, dz = pjx - pix, pjy - piy, pjz - piz
    f1 = _safenorm(dx, dy, dz)
    f2 = _angle(nix, niy, niz, dx, dy, dz)
    f3 = _angle(njx, njy, njz, dx, dy, dz)
    f4 = _angle(nix, niy, niz, njx, njy, njz)
    z = jnp.zeros_like(f1)
    o_ref[...] = jnp.stack([f1, f2, f3, f4, z, z, z, z], axis=0)


def _ppf(comp):
    return pl.pallas_call(
        _ppf_body,
        grid=(_TR // 16,),
        in_specs=[pl.BlockSpec((12, 16, 128), lambda i: (0, i, 0))],
        out_specs=pl.BlockSpec((8, 16, 128), lambda i: (0, i, 0)),
        out_shape=jax.ShapeDtypeStruct((8, _TR, 128), jnp.float32),
    )(comp)


def _b1_body(us_ref, pf_ref, w_ref, b_ref, h_ref, st_ref):
    i = pl.program_id(0)

    @pl.when(i == 0)
    def _():
        st_ref[...] = jnp.zeros_like(st_ref)

    h = us_ref[...] + jnp.dot(w_ref[...], pf_ref[...],
                              preferred_element_type=jnp.float32) + b_ref[...]
    h_ref[...] = h
    gid = i * _BLK + lax.broadcasted_iota(jnp.int32, (1, _BLK), 1)
    m = (gid < _E).astype(jnp.float32)
    hm = h * m
    st_ref[0:32, 0:1] = st_ref[0:32, 0:1] + jnp.sum(hm, axis=1, keepdims=True)
    st_ref[0:32, 1:2] = st_ref[0:32, 1:2] + jnp.sum(hm * h, axis=1,
                                                    keepdims=True)


def _b1(us_t, pf8, w1pt, b1c):
    return pl.pallas_call(
        _b1_body,
        grid=(_NBLK,),
        in_specs=[
            pl.BlockSpec((32, _BLK), lambda i: (0, i)),
            pl.BlockSpec((8, _BLK), lambda i: (0, i)),
            pl.BlockSpec((32, 8), lambda i: (0, 0)),
            pl.BlockSpec((32, 1), lambda i: (0, 0)),
        ],
        out_specs=[
            pl.BlockSpec((32, _BLK), lambda i: (0, i)),
            pl.BlockSpec((32, 128), lambda i: (0, 0)),
        ],
        out_shape=[
            jax.ShapeDtypeStruct((32, _EP), jnp.float32),
            jax.ShapeDtypeStruct((32, 128), jnp.float32),
        ],
    )(us_t, pf8, w1pt, b1c)


def _b2_body(h_ref, st_in, g_ref, be_ref, w_ref, b_ref, h2_ref, st_ref):
    i = pl.program_id(0)

    @pl.when(i == 0)
    def _():
        st_ref[...] = jnp.zeros_like(st_ref)

    mu = st_in[0:32, 0:1] / _E
    var = st_in[0:32, 1:2] / _E - mu * mu
    rstd = lax.rsqrt(var + _EPS)
    a = (h_ref[...] - mu) * rstd * g_ref[...] + be_ref[...]
    a = jnp.maximum(a, 0.0)
    h2 = jnp.dot(w_ref[...], a, preferred_element_type=jnp.float32) + b_ref[...]
    h2_ref[...] = h2
    gid = i * _BLK + lax.broadcasted_iota(jnp.int32, (1, _BLK), 1)
    m = (gid < _E).astype(jnp.float32)
    hm = h2 * m
    st_ref[0:32, 0:1] = st_ref[0:32, 0:1] + jnp.sum(hm, axis=1, keepdims=True)
    st_ref[0:32, 1:2] = st_ref[0:32, 1:2] + jnp.sum(hm * h2, axis=1,
                                                    keepdims=True)


def _b2(h1_t, st1, g1c, be1c, w2t, b2c):
    return pl.pallas_call(
        _b2_body,
        grid=(_NBLK,),
        in_specs=[
            pl.BlockSpec((32, _BLK), lambda i: (0, i)),
            pl.BlockSpec((32, 128), lambda i: (0, 0)),
            pl.BlockSpec((32, 1), lambda i: (0, 0)),
            pl.BlockSpec((32, 1), lambda i: (0, 0)),
            pl.BlockSpec((32, 32), lambda i: (0, 0)),
            pl.BlockSpec((32, 1), lambda i: (0, 0)),
        ],
        out_specs=[
            pl.BlockSpec((32, _BLK), lambda i: (0, i)),
            pl.BlockSpec((32, 128), lambda i: (0, 0)),
        ],
        out_shape=[
            jax.ShapeDtypeStruct((32, _EP), jnp.float32),
            jax.ShapeDtypeStruct((32, 128), jnp.float32),
        ],
    )(h1_t, st1, g1c, be1c, w2t, b2c)


def _b3_body(h_ref, st_in, g_ref, be_ref, w_ref, b_ref, h3_ref):
    mu = st_in[0:32, 0:1] / _E
    var = st_in[0:32, 1:2] / _E - mu * mu
    rstd = lax.rsqrt(var + _EPS)
    a = (h_ref[...] - mu) * rstd * g_ref[...] + be_ref[...]
    a = jnp.maximum(a, 0.0)
    h3 = lax.dot_general(a, w_ref[...], (((0,), (0,)), ((), ())),
                         preferred_element_type=jnp.float32)
    h3_ref[...] = h3 + b_ref[...]


def _b3(h2_t, st2, g2c, be2c, w3, b3r):
    return pl.pallas_call(
        _b3_body,
        grid=(_NBLK,),
        in_specs=[
            pl.BlockSpec((32, _BLK), lambda i: (0, i)),
            pl.BlockSpec((32, 128), lambda i: (0, 0)),
            pl.BlockSpec((32, 1), lambda i: (0, 0)),
            pl.BlockSpec((32, 1), lambda i: (0, 0)),
            pl.BlockSpec((32, 128), lambda i: (0, 0)),
            pl.BlockSpec((1, 128), lambda i: (0, 0)),
        ],
        out_specs=pl.BlockSpec((_BLK, 128), lambda i: (i, 0)),
        out_shape=jax.ShapeDtypeStruct((_EP, 128), jnp.float32),
    )(h2_t, st2, g2c, be2c, w3, b3r)


@functools.lru_cache(maxsize=None)
def _build_segmax_sc():
    mesh = plsc.VectorSubcoreMesh(core_axis_name="c", subcore_axis_name="s")

    @functools.partial(
        pl.kernel,
        mesh=mesh,
        out_type=[
            jax.ShapeDtypeStruct((_NW, _CAP), jnp.int32),
            jax.ShapeDtypeStruct((_NW, _CAP), jnp.int32),
            jax.ShapeDtypeStruct((_NW, 16), jnp.int32),
        ],
        scratch_types=[
            pltpu.VMEM((_CAP,), jnp.int32),
            pltpu.VMEM((_CAP,), jnp.int32),
            pltpu.VMEM((16,), jnp.int32),
            pltpu.VMEM((_BLK,), jnp.int32),
            pltpu.VMEM((_BLK,), jnp.int32),
            pltpu.SemaphoreType.DMA,
            pltpu.SemaphoreType.DMA,
        ],
        compiler_params=_sc_params,
    )
    def _compact_sc(dst_hbm, dlists_hbm, elists_hbm, counts_hbm,
                    dl, el, cnt_v, dc0, dc1, sem2, sem3):
        wid = lax.axis_index("s") * 2 + lax.axis_index("c")
        lo = wid * _ROWS_W

        limit = jnp.uint32(_ROWS_W)
        iota16 = lax.iota(jnp.int32, 16)

        def _dissue(t, dc, sem):
            tc = jnp.minimum(t, _NBLK - 1)
            return pltpu.async_copy(dst_hbm.at[pl.ds(tc * _BLK, _BLK)],
                                    dc, sem)

        def _dwait(dc, sem):
            pltpu.make_async_copy(dst_hbm.at[pl.ds(0, _BLK)], dc, sem).wait()

        def _scan_chunk(t, dc, n):
            def body(v, n):
                base = t * _BLK + v * 64
                ds_ = [dc[pl.ds(v * 64 + 16 * k, 16)] for k in range(4)]
                ms = [plsc.bitcast(d - lo, jnp.uint32) < limit for d in ds_]
                cs = [plsc.all_reduce_population_count(m)[0] for m in ms]
                for k in range(4):
                    ev = base + 16 * k + iota16
                    plsc.store_compressed(dl.at[pl.ds(n, 16)], ds_[k],
                                          mask=ms[k])
                    plsc.store_compressed(el.at[pl.ds(n, 16)], ev,
                                          mask=ms[k])
                    n = n + cs[k]
                return n

            return lax.fori_loop(0, _BLK // 64, body, n)

        _dissue_h = _dissue(0, dc0, sem2)
        _dissue_h = _dissue(1, dc1, sem3)

        def _scan_pair(q, n):
            _dwait(dc0, sem2)
            n = _scan_chunk(2 * q, dc0, n)
            _dissue(2 * q + 2, dc0, sem2)
            _dwait(dc1, sem3)
            n = _scan_chunk(2 * q + 1, dc1, n)
            _dissue(2 * q + 3, dc1, sem3)
            return n

        n = lax.fori_loop(0, _NBLK // 2, _scan_pair, 0)
        _dwait(dc0, sem2)
        _dwait(dc1, sem3)

        dummy_d = jnp.full((16,), lo + _ROWS_W, jnp.int32)
        dummy_e = jnp.zeros((16,), jnp.int32)
        tail_idx = n + lax.iota(jnp.int32, 16)
        okm = tail_idx < _CAP
        plsc.store_scatter(dl, [jnp.where(okm, tail_idx, _CAP - 1)],
                           dummy_d, mask=okm)
        plsc.store_scatter(el, [jnp.where(okm, tail_idx, _CAP - 1)],
                           dummy_e, mask=okm)
        npad = (n + 2 * _CH - 1) // (2 * _CH) * (2 * _CH)

        def _padv(q, _):
            dl[pl.ds(q * 16, 16)] = dummy_d
            el[pl.ds(q * 16, 16)] = dummy_e
            return _

        lax.fori_loop(n // 16 + 1, npad // 16, _padv, 0)

        cnt_v[pl.ds(0, 16)] = jnp.full((16,), 1, jnp.int32) * npad
        pltpu.sync_copy(dl, dlists_hbm.at[wid])
        pltpu.sync_copy(el, elists_hbm.at[wid])
        pltpu.sync_copy(cnt_v, counts_hbm.at[wid])

    @functools.partial(
        pl.kernel,
        mesh=mesh,
        out_type=jax.ShapeDtypeStruct((_NP, 128), jnp.float32),
        scratch_types=[
            pltpu.VMEM((_ACC_R, 128), jnp.float32),
            pltpu.VMEM((_CAP,), jnp.int32),
            pltpu.VMEM((_CAP,), jnp.int32),
            pltpu.VMEM((16,), jnp.int32),
            pltpu.VMEM((_CH, 128), jnp.float32),
            pltpu.VMEM((_CH, 128), jnp.float32),
            pltpu.SemaphoreType.DMA,
            pltpu.SemaphoreType.DMA,
        ],
        compiler_params=_sc_params,
    )
    def _segmax_sc(h3_hbm, dlists_hbm, elists_hbm, counts_hbm, out_hbm,
                   acc, dl, el, cnt_v, rb0, rb1, sem0, sem1):
        wid = lax.axis_index("s") * 2 + lax.axis_index("c")
        lo = wid * _ROWS_W
        neg = jnp.full((16,), _NEG, jnp.float32)

        h_dl = pltpu.async_copy(dlists_hbm.at[wid], dl, sem0)
        h_el = pltpu.async_copy(elists_hbm.at[wid], el, sem1)
        pltpu.sync_copy(counts_hbm.at[wid], cnt_v)
        npad = cnt_v[pl.ds(0, 16)][0]

        @pl.loop(0, _ACC_R)
        def _init(r):
            for cc in range(8):
                acc[r, pl.ds(cc * 16, 16)] = neg

        h_dl.wait()
        h_el.wait()

        nb = npad // _CH

        def _issue(b, rb, sem):
            bc = jnp.minimum(b, nb - 1)
            return pltpu.async_copy(
                h3_hbm.at[el.at[pl.ds(bc * _CH, _CH)]], rb, sem)

        def _wait(rb, sem):
            pltpu.make_async_copy(
                h3_hbm.at[el.at[pl.ds(0, _CH)]], rb, sem).wait()

        def _rmw(b, rb):
            def grp(gi, _):
                dv16 = dl[pl.ds(b * _CH + gi * 16, 16)] - lo
                for j in range(16):
                    l = dv16[j]
                    for r in range(8):
                        a = acc[l, pl.ds(r * 16, 16)]
                        v = rb[gi * 16 + j, pl.ds(r * 16, 16)]
                        acc[l, pl.ds(r * 16, 16)] = jnp.maximum(a, v)
                return _

            lax.fori_loop(0, _CH // 16, grp, 0)

        _issue(0, rb0, sem0)
        _issue(1, rb1, sem1)

        def _pair(p, _):
            _wait(rb0, sem0)
            _rmw(2 * p, rb0)
            _issue(2 * p + 2, rb0, sem0)
            _wait(rb1, sem1)
            _rmw(2 * p + 1, rb1)
            _issue(2 * p + 3, rb1, sem1)
            return _

        lax.fori_loop(0, nb // 2, _pair, 0)
        _wait(rb0, sem0)
        _wait(rb1, sem1)

        pltpu.sync_copy(acc.at[pl.ds(0, _ROWS_W)],
                        out_hbm.at[pl.ds(lo, _ROWS_W)])

    return _compact_sc, _segmax_sc


def _sm_body(o_ref, r_ref):
    x = o_ref[...]
    rid = lax.broadcasted_iota(jnp.int32, (_NP, 1), 0)
    m = rid < _N
    xm = jnp.where(m, x, _NEG)
    cmax = jnp.max(xm, axis=0, keepdims=True)
    e = jnp.where(m, jnp.exp(xm - cmax), 0.0)
    s = jnp.sum(e, axis=0, keepdims=True)
    r_ref[...] = (e / s)[0:_N, :]


def _softmax(outp):
    return pl.pallas_call(
        _sm_body,
        out_shape=jax.ShapeDtypeStruct((_N, 128), jnp.float32),
    )(outp)


def kernel(x, pos, norm, edge_index, W1, b1, g1, be1, W2, b2, g2, be2, W3, b3):
    ei = edge_index.astype(jnp.int32)
    loops = jnp.arange(_N, dtype=jnp.int32)
    src = jnp.concatenate([ei[0], loops,
                           jnp.zeros((_EP - _E,), jnp.int32)])
    dst = jnp.concatenate([ei[1], loops,
                           jnp.full((_EP - _E,), _NP - 1, jnp.int32)])
    g = jnp.zeros((_NP, 8), jnp.float32)
    g = g.at[:_N, 0:3].set(pos).at[:_N, 3:6].set(norm).reshape(_NP * 8)
    w1x = W1[:128]
    w1pt = jnp.zeros((32, 8), jnp.float32).at[:, 0:4].set(W1[128:132].T)

    compact_sc, segmax_sc = _build_segmax_sc()
    u = _node_proj(x, w1x)
    comp, us = _build_gather_sc()(g, u, src, dst)
    dlists, elists, counts = compact_sc(dst)
    ppf8 = _ppf(comp).reshape(8, _EP)
    us_t = us.T
    h1_t, st1 = _b1(us_t, ppf8, w1pt, b1[:, None])
    h2_t, st2 = _b2(h1_t, st1, g1[:, None], be1[:, None], W2.T, b2[:, None])
    h3 = _b3(h2_t, st2, g2[:, None], be2[:, None], W3, b3[None, :])
    outp = segmax_sc(h3, dlists, elists, counts)
    return _softmax(outp)


# speedup vs baseline: 5.2504x; 1.0457x over previous
import dataclasses
import functools

import jax
import jax.numpy as jnp
from jax import lax
from jax.experimental import pallas as pl
from jax.experimental.pallas import tpu as pltpu
from jax.experimental.pallas import tpu_sc as plsc

_N = 10000
_NP = 10240
_E = 330000
_BLK = 2048
_NBLK = 162
_EP = _BLK * _NBLK
_NW = 32
_EW = _EP // _NW
_CH = 128
_NCH = _EW // _CH
_TR = _EP // 128
_ROWS_W = _NP // _NW
_ACC_R = _ROWS_W + 8
_CAP = 16384
_EPS = 1e-5
_NEG = -3.0e38

_sc_params = pltpu.CompilerParams()
if "needs_layout_passes" in pltpu.CompilerParams.__dataclass_fields__:
    _sc_params = dataclasses.replace(_sc_params, needs_layout_passes=False)
if "use_tc_tiling_on_sc" in pltpu.CompilerParams.__dataclass_fields__:
    _sc_params = dataclasses.replace(_sc_params, use_tc_tiling_on_sc=False)


def _u_body(x_ref, w_ref, o_ref):
    o_ref[...] = jnp.dot(x_ref[...], w_ref[...],
                         preferred_element_type=jnp.float32)


def _node_proj(x, w1x):
    return pl.pallas_call(
        _u_body,
        out_shape=jax.ShapeDtypeStruct((_N, 32), jnp.float32),
    )(x, w1x)


@functools.lru_cache(maxsize=None)
def _build_gather_sc():
    mesh = plsc.VectorSubcoreMesh(core_axis_name="c", subcore_axis_name="s")

    @functools.partial(
        pl.kernel,
        mesh=mesh,
        out_type=[
            jax.ShapeDtypeStruct((12, _TR, 128), jnp.float32),
            jax.ShapeDtypeStruct((_EP, 32), jnp.float32),
        ],
        scratch_types=[
            pltpu.VMEM((_NP * 8,), jnp.float32),
            pltpu.VMEM((_CH,), jnp.int32),
            pltpu.VMEM((_CH,), jnp.int32),
            pltpu.VMEM((_CH,), jnp.int32),
            pltpu.VMEM((_CH,), jnp.int32),
            pltpu.VMEM((12, 1, 128), jnp.float32),
            pltpu.VMEM((12, 1, 128), jnp.float32),
            pltpu.VMEM((_CH, 32), jnp.float32),
            pltpu.VMEM((_CH, 32), jnp.float32),
            pltpu.SemaphoreType.DMA,
            pltpu.SemaphoreType.DMA,
            pltpu.SemaphoreType.DMA,
            pltpu.SemaphoreType.DMA,
            pltpu.SemaphoreType.DMA,
            pltpu.SemaphoreType.DMA,
            pltpu.SemaphoreType.DMA,
            pltpu.SemaphoreType.DMA,
        ],
        compiler_params=_sc_params,
    )
    def _gather_sc(g_hbm, u_hbm, src_hbm, dst_hbm, comp_hbm, us_hbm,
                   gv, sv0, sv1, dv0, dv1, cb0, cb1, ub0, ub1,
                   sd0, sd1, su0, su1, sc0, sc1, so0, so1):
        wid = lax.axis_index("s") * 2 + lax.axis_index("c")
        base = wid * _EW
        pltpu.sync_copy(g_hbm, gv)

        sets = [(sv0, dv0, cb0, ub0, sd0, su0, sc0, so0),
                (sv1, dv1, cb1, ub1, sd1, su1, sc1, so1)]

        def _eb(t):
            return base + jnp.minimum(t, _NCH - 1) * _CH

        def _issue_idx(t, s):
            sv, dv, _, _, sd, _, _, _ = s
            eb = _eb(t)
            pltpu.async_copy(src_hbm.at[pl.ds(eb, _CH)], sv, sd)
            pltpu.async_copy(dst_hbm.at[pl.ds(eb, _CH)], dv, sd)

        def _issue_out(t, s):
            _, _, cb, ub, _, _, sc, so = s
            eb = _eb(t)
            pltpu.async_copy(cb, comp_hbm.at[:, pl.ds(eb // 128, 1), :], sc)
            pltpu.async_copy(ub, us_hbm.at[pl.ds(eb, _CH)], so)

        def _wait(src_like, dst_like, sem):
            pltpu.make_async_copy(src_like, dst_like, sem).wait()

        for b in range(2):
            _issue_idx(b, sets[b])
            _issue_out(b, sets[b])

        def _half(t, s):
            sv, dv, cb, ub, sd, su, sc, so = s
            eb = _eb(t)
            _wait(src_hbm.at[pl.ds(0, _CH)], sv, sd)
            _wait(src_hbm.at[pl.ds(0, _CH)], dv, sd)
            _wait(ub, us_hbm.at[pl.ds(0, _CH)], so)
            pltpu.async_copy(u_hbm.at[sv], ub, su)
            _wait(cb, comp_hbm.at[:, pl.ds(0, 1), :], sc)

            @pl.loop(0, _CH // 16)
            def _vec(v):
                si = sv[pl.ds(v * 16, 16)] * 8
                di = dv[pl.ds(v * 16, 16)] * 8
                for k in range(6):
                    cb[k, 0, pl.ds(v * 16, 16)] = plsc.load_gather(gv, [si + k])
                    cb[6 + k, 0, pl.ds(v * 16, 16)] = plsc.load_gather(
                        gv, [di + k])

            pltpu.async_copy(cb, comp_hbm.at[:, pl.ds(eb // 128, 1), :], sc)
            _wait(u_hbm.at[sv], ub, su)
            pltpu.async_copy(ub, us_hbm.at[pl.ds(eb, _CH)], so)
            _issue_idx(t + 2, s)

        def _pair(q, _):
            _half(2 * q, sets[0])
            _half(2 * q + 1, sets[1])
            return _

        lax.fori_loop(0, (_NCH + 1) // 2, _pair, 0)
        for b in range(2):
            sv, dv, cb, ub, sd, su, sc, so = sets[b]
            _wait(src_hbm.at[pl.ds(0, _CH)], sv, sd)
            _wait(src_hbm.at[pl.ds(0, _CH)], dv, sd)
            _wait(cb, comp_hbm.at[:, pl.ds(0, 1), :], sc)
            _wait(ub, us_hbm.at[pl.ds(0, _CH)], so)

    return _gather_sc


def _safenorm(x, y, z):
    sq = x * x + y * y + z * z
    return jnp.where(sq > 0, jnp.sqrt(jnp.where(sq > 0, sq, 1.0)), 0.0)


def _angle(ax, ay, az, bx, by, bz):
    cx = ay * bz - az * by
    cy = az * bx - ax * bz
    cz = ax * by - ay * bx
    dot = ax * bx + ay * by + az * bz
    dot = jnp.where(dot == 0.0, 0.0, dot)
    return jnp.arctan2(_safenorm(cx, cy, cz), dot)


def _ppf_body(c_ref, o_ref):
    c = c_ref[...]
    pjx, pjy, pjz = c[0], c[1], c[2]
    njx, njy, njz = c[3], c[4], c[5]
    pix, piy, piz = c[6], c[7], c[8]
    nix, niy, niz = c[9], c[10], c[11]
    dx, dy, dz = pjx - pix, pjy - piy, pjz - piz
    f1 = _safenorm(dx, dy, dz)
    f2 = _angle(nix, niy, niz, dx, dy, dz)
    f3 = _angle(njx, njy, njz, dx, dy, dz)
    f4 = _angle(nix, niy, niz, njx, njy, njz)
    z = jnp.zeros_like(f1)
    o_ref[...] = jnp.stack([f1, f2, f3, f4, z, z, z, z], axis=0)


def _ppf(comp):
    return pl.pallas_call(
        _ppf_body,
        grid=(_TR // 16,),
        in_specs=[pl.BlockSpec((12, 16, 128), lambda i: (0, i, 0))],
        out_specs=pl.BlockSpec((8, 16, 128), lambda i: (0, i, 0)),
        out_shape=jax.ShapeDtypeStruct((8, _TR, 128), jnp.float32),
    )(comp)


def _b1_body(us_ref, pf_ref, w_ref, b_ref, h_ref, st_ref):
    i = pl.program_id(0)

    @pl.when(i == 0)
    def _():
        st_ref[...] = jnp.zeros_like(st_ref)

    h = us_ref[...] + jnp.dot(w_ref[...], pf_ref[...],
                              preferred_element_type=jnp.float32) + b_ref[...]
    h_ref[...] = h
    gid = i * _BLK + lax.broadcasted_iota(jnp.int32, (1, _BLK), 1)
    m = (gid < _E).astype(jnp.float32)
    hm = h * m
    st_ref[0:32, 0:1] = st_ref[0:32, 0:1] + jnp.sum(hm, axis=1, keepdims=True)
    st_ref[0:32, 1:2] = st_ref[0:32, 1:2] + jnp.sum(hm * h, axis=1,
                                                    keepdims=True)


def _b1(us_t, pf8, w1pt, b1c):
    return pl.pallas_call(
        _b1_body,
        grid=(_NBLK,),
        in_specs=[
            pl.BlockSpec((32, _BLK), lambda i: (0, i)),
            pl.BlockSpec((8, _BLK), lambda i: (0, i)),
            pl.BlockSpec((32, 8), lambda i: (0, 0)),
            pl.BlockSpec((32, 1), lambda i: (0, 0)),
        ],
        out_specs=[
            pl.BlockSpec((32, _BLK), lambda i: (0, i)),
            pl.BlockSpec((32, 128), lambda i: (0, 0)),
        ],
        out_shape=[
            jax.ShapeDtypeStruct((32, _EP), jnp.float32),
            jax.ShapeDtypeStruct((32, 128), jnp.float32),
        ],
    )(us_t, pf8, w1pt, b1c)


def _b2_body(h_ref, st_in, g_ref, be_ref, w_ref, b_ref, h2_ref, st_ref):
    i = pl.program_id(0)

    @pl.when(i == 0)
    def _():
        st_ref[...] = jnp.zeros_like(st_ref)

    mu = st_in[0:32, 0:1] / _E
    var = st_in[0:32, 1:2] / _E - mu * mu
    rstd = lax.rsqrt(var + _EPS)
    a = (h_ref[...] - mu) * rstd * g_ref[...] + be_ref[...]
    a = jnp.maximum(a, 0.0)
    h2 = jnp.dot(w_ref[...], a, preferred_element_type=jnp.float32) + b_ref[...]
    h2_ref[...] = h2
    gid = i * _BLK + lax.broadcasted_iota(jnp.int32, (1, _BLK), 1)
    m = (gid < _E).astype(jnp.float32)
    hm = h2 * m
    st_ref[0:32, 0:1] = st_ref[0:32, 0:1] + jnp.sum(hm, axis=1, keepdims=True)
    st_ref[0:32, 1:2] = st_ref[0:32, 1:2] + jnp.sum(hm * h2, axis=1,
                                                    keepdims=True)


def _b2(h1_t, st1, g1c, be1c, w2t, b2c):
    return pl.pallas_call(
        _b2_body,
        grid=(_NBLK,),
        in_specs=[
            pl.BlockSpec((32, _BLK), lambda i: (0, i)),
            pl.BlockSpec((32, 128), lambda i: (0, 0)),
            pl.BlockSpec((32, 1), lambda i: (0, 0)),
            pl.BlockSpec((32, 1), lambda i: (0, 0)),
            pl.BlockSpec((32, 32), lambda i: (0, 0)),
            pl.BlockSpec((32, 1), lambda i: (0, 0)),
        ],
        out_specs=[
            pl.BlockSpec((32, _BLK), lambda i: (0, i)),
            pl.BlockSpec((32, 128), lambda i: (0, 0)),
        ],
        out_shape=[
            jax.ShapeDtypeStruct((32, _EP), jnp.float32),
            jax.ShapeDtypeStruct((32, 128), jnp.float32),
        ],
    )(h1_t, st1, g1c, be1c, w2t, b2c)


def _b3_body(h_ref, st_in, g_ref, be_ref, w_ref, b_ref, h3_ref):
    mu = st_in[0:32, 0:1] / _E
    var = st_in[0:32, 1:2] / _E - mu * mu
    rstd = lax.rsqrt(var + _EPS)
    a = (h_ref[...] - mu) * rstd * g_ref[...] + be_ref[...]
    a = jnp.maximum(a, 0.0)
    h3 = lax.dot_general(a, w_ref[...], (((0,), (0,)), ((), ())),
                         preferred_element_type=jnp.float32)
    h3_ref[...] = h3 + b_ref[...]


def _b3(h2_t, st2, g2c, be2c, w3, b3r):
    return pl.pallas_call(
        _b3_body,
        grid=(_NBLK,),
        in_specs=[
            pl.BlockSpec((32, _BLK), lambda i: (0, i)),
            pl.BlockSpec((32, 128), lambda i: (0, 0)),
            pl.BlockSpec((32, 1), lambda i: (0, 0)),
            pl.BlockSpec((32, 1), lambda i: (0, 0)),
            pl.BlockSpec((32, 128), lambda i: (0, 0)),
            pl.BlockSpec((1, 128), lambda i: (0, 0)),
        ],
        out_specs=pl.BlockSpec((_BLK, 128), lambda i: (i, 0)),
        out_shape=jax.ShapeDtypeStruct((_EP, 128), jnp.float32),
    )(h2_t, st2, g2c, be2c, w3, b3r)


@functools.lru_cache(maxsize=None)
def _build_segmax_sc():
    mesh = plsc.VectorSubcoreMesh(core_axis_name="c", subcore_axis_name="s")

    @functools.partial(
        pl.kernel,
        mesh=mesh,
        out_type=[
            jax.ShapeDtypeStruct((_NW, _CAP), jnp.int32),
            jax.ShapeDtypeStruct((_NW, _CAP), jnp.int32),
            jax.ShapeDtypeStruct((_NW, 16), jnp.int32),
        ],
        scratch_types=[
            pltpu.VMEM((_CAP,), jnp.int32),
            pltpu.VMEM((_CAP,), jnp.int32),
            pltpu.VMEM((16,), jnp.int32),
            pltpu.VMEM((_BLK,), jnp.int32),
            pltpu.VMEM((_BLK,), jnp.int32),
            pltpu.SemaphoreType.DMA,
            pltpu.SemaphoreType.DMA,
        ],
        compiler_params=_sc_params,
    )
    def _compact_sc(dst_hbm, dlists_hbm, elists_hbm, counts_hbm,
                    dl, el, cnt_v, dc0, dc1, sem2, sem3):
        wid = lax.axis_index("s") * 2 + lax.axis_index("c")
        lo = wid * _ROWS_W

        limit = jnp.uint32(_ROWS_W)
        iota16 = lax.iota(jnp.int32, 16)

        def _dissue(t, dc, sem):
            tc = jnp.minimum(t, _NBLK - 1)
            return pltpu.async_copy(dst_hbm.at[pl.ds(tc * _BLK, _BLK)],
                                    dc, sem)

        def _dwait(dc, sem):
            pltpu.make_async_copy(dst_hbm.at[pl.ds(0, _BLK)], dc, sem).wait()

        def _scan_chunk(t, dc, n):
            def body(v, n):
                base = t * _BLK + v * 64
                ds_ = [dc[pl.ds(v * 64 + 16 * k, 16)] for k in range(4)]
                ms = [plsc.bitcast(d - lo, jnp.uint32) < limit for d in ds_]
                cs = [plsc.all_reduce_population_count(m)[0] for m in ms]
                for k in range(4):
                    ev = base + 16 * k + iota16
                    plsc.store_compressed(dl.at[pl.ds(n, 16)], ds_[k],
                                          mask=ms[k])
                    plsc.store_compressed(el.at[pl.ds(n, 16)], ev,
                                          mask=ms[k])
                    n = n + cs[k]
                return n

            return lax.fori_loop(0, _BLK // 64, body, n)

        _dissue_h = _dissue(0, dc0, sem2)
        _dissue_h = _dissue(1, dc1, sem3)

        def _scan_pair(q, n):
            _dwait(dc0, sem2)
            n = _scan_chunk(2 * q, dc0, n)
            _dissue(2 * q + 2, dc0, sem2)
            _dwait(dc1, sem3)
            n = _scan_chunk(2 * q + 1, dc1, n)
            _dissue(2 * q + 3, dc1, sem3)
            return n

        n = lax.fori_loop(0, _NBLK // 2, _scan_pair, 0)
        _dwait(dc0, sem2)
        _dwait(dc1, sem3)

        dummy_d = jnp.full((16,), lo + _ROWS_W, jnp.int32)
        dummy_e = jnp.zeros((16,), jnp.int32)
        tail_idx = n + lax.iota(jnp.int32, 16)
        okm = tail_idx < _CAP
        plsc.store_scatter(dl, [jnp.where(okm, tail_idx, _CAP - 1)],
                           dummy_d, mask=okm)
        plsc.store_scatter(el, [jnp.where(okm, tail_idx, _CAP - 1)],
                           dummy_e, mask=okm)
        npad = (n + 2 * _CH - 1) // (2 * _CH) * (2 * _CH)

        def _padv(q, _):
            dl[pl.ds(q * 16, 16)] = dummy_d
            el[pl.ds(q * 16, 16)] = dummy_e
            return _

        lax.fori_loop(n // 16 + 1, npad // 16, _padv, 0)

        cnt_v[pl.ds(0, 16)] = jnp.full((16,), 1, jnp.int32) * npad
        pltpu.sync_copy(dl, dlists_hbm.at[wid])
        pltpu.sync_copy(el, elists_hbm.at[wid])
        pltpu.sync_copy(cnt_v, counts_hbm.at[wid])

    @functools.partial(
        pl.kernel,
        mesh=mesh,
        out_type=jax.ShapeDtypeStruct((_NP, 128), jnp.float32),
        scratch_types=[
            pltpu.VMEM((_ACC_R, 128), jnp.float32),
            pltpu.VMEM((_CAP,), jnp.int32),
            pltpu.VMEM((_CAP,), jnp.int32),
            pltpu.VMEM((16,), jnp.int32),
            pltpu.VMEM((_CH, 128), jnp.float32),
            pltpu.VMEM((_CH, 128), jnp.float32),
            pltpu.SemaphoreType.DMA,
            pltpu.SemaphoreType.DMA,
        ],
        compiler_params=_sc_params,
    )
    def _segmax_sc(h3_hbm, dlists_hbm, elists_hbm, counts_hbm, out_hbm,
                   acc, dl, el, cnt_v, rb0, rb1, sem0, sem1):
        wid = lax.axis_index("s") * 2 + lax.axis_index("c")
        lo = wid * _ROWS_W
        neg = jnp.full((16,), _NEG, jnp.float32)

        h_dl = pltpu.async_copy(dlists_hbm.at[wid], dl, sem0)
        h_el = pltpu.async_copy(elists_hbm.at[wid], el, sem1)
        pltpu.sync_copy(counts_hbm.at[wid], cnt_v)
        npad = cnt_v[pl.ds(0, 16)][0]

        @pl.loop(0, _ACC_R)
        def _init(r):
            for cc in range(8):
                acc[r, pl.ds(cc * 16, 16)] = neg

        h_dl.wait()
        h_el.wait()

        nb = npad // _CH

        def _issue(b, rb, sem):
            bc = jnp.minimum(b, nb - 1)
            return pltpu.async_copy(
                h3_hbm.at[el.at[pl.ds(bc * _CH, _CH)]], rb, sem)

        def _wait(rb, sem):
            pltpu.make_async_copy(
                h3_hbm.at[el.at[pl.ds(0, _CH)]], rb, sem).wait()

        def _rmw(b, rb):
            def grp(gi, _):
                dv16 = dl[pl.ds(b * _CH + gi * 16, 16)] - lo
                ls = [dv16[j] for j in range(16)]
                for j in range(16):
                    l = ls[j]
                    for r in range(8):
                        a = acc[l, pl.ds(r * 16, 16)]
                        v = rb[gi * 16 + j, pl.ds(r * 16, 16)]
                        acc[l, pl.ds(r * 16, 16)] = jnp.maximum(a, v)
                return _

            lax.fori_loop(0, _CH // 16, grp, 0)

        _issue(0, rb0, sem0)
        _issue(1, rb1, sem1)

        def _pair(p, _):
            _wait(rb0, sem0)
            _rmw(2 * p, rb0)
            _issue(2 * p + 2, rb0, sem0)
            _wait(rb1, sem1)
            _rmw(2 * p + 1, rb1)
            _issue(2 * p + 3, rb1, sem1)
            return _

        lax.fori_loop(0, nb // 2, _pair, 0)
        _wait(rb0, sem0)
        _wait(rb1, sem1)

        pltpu.sync_copy(acc.at[pl.ds(0, _ROWS_W)],
                        out_hbm.at[pl.ds(lo, _ROWS_W)])

    return _compact_sc, _segmax_sc


def _sm_body(o_ref, r_ref):
    x = o_ref[...]
    rid = lax.broadcasted_iota(jnp.int32, (_NP, 1), 0)
    m = rid < _N
    xm = jnp.where(m, x, _NEG)
    cmax = jnp.max(xm, axis=0, keepdims=True)
    e = jnp.where(m, jnp.exp(xm - cmax), 0.0)
    s = jnp.sum(e, axis=0, keepdims=True)
    r_ref[...] = (e / s)[0:_N, :]


def _softmax(outp):
    return pl.pallas_call(
        _sm_body,
        out_shape=jax.ShapeDtypeStruct((_N, 128), jnp.float32),
    )(outp)


def kernel(x, pos, norm, edge_index, W1, b1, g1, be1, W2, b2, g2, be2, W3, b3):
    ei = edge_index.astype(jnp.int32)
    loops = jnp.arange(_N, dtype=jnp.int32)
    src = jnp.concatenate([ei[0], loops,
                           jnp.zeros((_EP - _E,), jnp.int32)])
    dst = jnp.concatenate([ei[1], loops,
                           jnp.full((_EP - _E,), _NP - 1, jnp.int32)])
    g = jnp.zeros((_NP, 8), jnp.float32)
    g = g.at[:_N, 0:3].set(pos).at[:_N, 3:6].set(norm).reshape(_NP * 8)
    w1x = W1[:128]
    w1pt = jnp.zeros((32, 8), jnp.float32).at[:, 0:4].set(W1[128:132].T)

    compact_sc, segmax_sc = _build_segmax_sc()
    u = _node_proj(x, w1x)
    comp, us = _build_gather_sc()(g, u, src, dst)
    dlists, elists, counts = compact_sc(dst)
    ppf8 = _ppf(comp).reshape(8, _EP)
    us_t = us.T
    h1_t, st1 = _b1(us_t, ppf8, w1pt, b1[:, None])
    h2_t, st2 = _b2(h1_t, st1, g1[:, None], be1[:, None], W2.T, b2[:, None])
    h3 = _b3(h2_t, st2, g2[:, None], be2[:, None], W3, b3[None, :])
    outp = segmax_sc(h3, dlists, elists, counts)
    return _softmax(outp)
